# Initial kernel scaffold; baseline (speedup 1.0000x reference)
import jax
import jax.numpy as jnp
from jax.experimental import pallas as pl


def kernel(x, pLDDT, bfactor, edge_index, batch, gn1_w, gn1_b, gn1_ms, W1, b1, gn2_w, gn2_b, gn2_ms, W2, b2, bn1_g, bn1_b, fc1_W, fc1_b, bn2_g, bn2_b, fc2_W, fc2_b):
    raise NotImplementedError("write your pallas kernel here")


# speedup vs baseline: 11.6236x; 11.6236x over previous
import functools

import jax
import jax.numpy as jnp
from jax import lax
from jax.experimental import pallas as pl
from jax.experimental.pallas import tpu as pltpu
from jax.experimental.pallas import tpu_sc as plsc

N = 10000
E = 320000
G = 16
DF = 128
EPS = 1e-5

NC = 2
NS = 16
NW = NC * NS
EPW = E // NW
K = 80
NCHUNK = EPW // K
RPT = 640
WB = 80
DEGW = 8

R = 1000
NB = N // R

_f32 = jnp.float32


def _sc_mesh():
    return plsc.VectorSubcoreMesh(
        core_axis_name="c", subcore_axis_name="s", num_cores=NC, num_subcores=NS)


def _make_sc_deg():
    @functools.partial(
        pl.kernel,
        out_type=jax.ShapeDtypeStruct((NC, N, DEGW), _f32),
        mesh=_sc_mesh(),
        compiler_params=pltpu.CompilerParams(use_tc_tiling_on_sc=False),
        scratch_types=[
            pltpu.VMEM((NCHUNK, K), jnp.int32),
            pltpu.VMEM((K, DEGW), _f32),
            pltpu.VMEM((WB, DEGW), _f32),
            pltpu.VMEM_SHARED((N, DEGW), _f32),
            pltpu.SemaphoreType.DMA,
        ],
    )
    def run(dst_hbm, ones_hbm, zeros_hbm, out_hbm, dstv, ones_v, wbuf, acc, sem):
        c = lax.axis_index("c")
        s = lax.axis_index("s")
        wid = c * NS + s
        nwb = jnp.where(s == NS - 1, 5, 8)
        pltpu.sync_copy(dst_hbm.at[wid], dstv)
        pltpu.sync_copy(ones_hbm, ones_v)
        pltpu.sync_copy(zeros_hbm, wbuf)

        def zacc(k, _):
            pltpu.sync_copy(wbuf, acc.at[pl.ds(s * RPT + k * WB, WB)])
            return 0

        lax.fori_loop(0, nwb, zacc, 0)
        plsc.subcore_barrier()

        def chunk(ci, _):
            pltpu.sync_copy(ones_v, acc.at[dstv.at[ci]], add=True)
            return 0

        lax.fori_loop(0, NCHUNK, chunk, 0)
        plsc.subcore_barrier()

        def wback(k, _):
            rows = pl.ds(s * RPT + k * WB, WB)
            pltpu.sync_copy(acc.at[rows], wbuf)
            pltpu.sync_copy(wbuf, out_hbm.at[c].at[rows])
            return 0

        lax.fori_loop(0, nwb, wback, 0)

    return run


DH = DF // 2


def _make_sc_agg():
    @functools.partial(
        pl.kernel,
        out_type=jax.ShapeDtypeStruct((NC, N, DH), _f32),
        mesh=_sc_mesh(),
        compiler_params=pltpu.CompilerParams(use_tc_tiling_on_sc=False),
        scratch_types=[
            pltpu.VMEM((NCHUNK, K), jnp.int32),
            pltpu.VMEM((NCHUNK, K), jnp.int32),
            pltpu.VMEM((K, DH), _f32),
            pltpu.VMEM((WB, DH), _f32),
            pltpu.VMEM_SHARED((N, DH), _f32),
            pltpu.SemaphoreType.DMA,
        ],
    )
    def run(xws_hbm, src_hbm, dst_hbm, out_hbm, srcv, dstv, gbuf, wbuf, acc, sem):
        c = lax.axis_index("c")
        s = lax.axis_index("s")
        wid = c * NS + s
        nwb = jnp.where(s == NS - 1, 5, 8)
        pltpu.sync_copy(src_hbm.at[wid], srcv)
        pltpu.sync_copy(dst_hbm.at[wid], dstv)

        zero16 = jnp.zeros((16,), _f32)

        def fill_zero(r, _):
            for j in range(DH // 16):
                wbuf[r, pl.ds(j * 16, 16)] = zero16
            return 0

        lax.fori_loop(0, WB, fill_zero, 0)

        def zacc(k, _):
            pltpu.sync_copy(wbuf, acc.at[pl.ds(s * RPT + k * WB, WB)])
            return 0

        lax.fori_loop(0, nwb, zacc, 0)
        plsc.subcore_barrier()

        def chunk(ci, _):
            pltpu.async_copy(xws_hbm.at[srcv.at[ci]], gbuf, sem).wait()
            pltpu.sync_copy(gbuf, acc.at[dstv.at[ci]], add=True)
            return 0

        lax.fori_loop(0, NCHUNK, chunk, 0)
        plsc.subcore_barrier()

        def wback(k, _):
            rows = pl.ds(s * RPT + k * WB, WB)
            pltpu.sync_copy(acc.at[rows], wbuf)
            pltpu.sync_copy(wbuf, out_hbm.at[c].at[rows])
            return 0

        lax.fori_loop(0, nwb, wback, 0)

    return run


_SC_DEG = _make_sc_deg()
_SC_AGG = _make_sc_agg()


def _sc_deg(dst3):
    return _SC_DEG(dst3, jnp.ones((K, DEGW), _f32), jnp.zeros((WB, DEGW), _f32))


def _sc_agg(xws, src3, dst3):
    lo = _SC_AGG(xws[:, :DH], src3, dst3)
    hi = _SC_AGG(xws[:, DH:], src3, dst3)
    return jnp.concatenate([lo, hi], axis=2)


def _onehot(bb, rows):
    return (bb == lax.broadcasted_iota(jnp.int32, (rows, G), 1)).astype(_f32)


def _leaky(x):
    return jnp.where(x >= 0, x, 0.01 * x)


def _tc_stats(xfull, batch2):
    D = xfull.shape[1]

    def body(x_ref, b_ref, s1_ref, s2_ref, cnt_ref):
        i = pl.program_id(0)

        @pl.when(i == 0)
        def _():
            s1_ref[...] = jnp.zeros_like(s1_ref)
            s2_ref[...] = jnp.zeros_like(s2_ref)
            cnt_ref[...] = jnp.zeros_like(cnt_ref)

        xb = x_ref[...]
        oh = _onehot(b_ref[...], R)
        dn = (((0,), (0,)), ((), ()))
        s1_ref[...] += lax.dot_general(oh, xb, dn, preferred_element_type=_f32, precision=lax.Precision.HIGHEST)
        s2_ref[...] += lax.dot_general(oh, xb * xb, dn, preferred_element_type=_f32, precision=lax.Precision.HIGHEST)
        cnt_ref[...] += jnp.broadcast_to(jnp.sum(oh, axis=0)[:, None], (G, 128))

    return pl.pallas_call(
        body,
        grid=(NB,),
        in_specs=[
            pl.BlockSpec((R, D), lambda i: (i, 0)),
            pl.BlockSpec((R, 1), lambda i: (i, 0)),
        ],
        out_specs=[
            pl.BlockSpec((G, D), lambda i: (0, 0)),
            pl.BlockSpec((G, D), lambda i: (0, 0)),
            pl.BlockSpec((G, 128), lambda i: (0, 0)),
        ],
        out_shape=[
            jax.ShapeDtypeStruct((G, D), _f32),
            jax.ShapeDtypeStruct((G, D), _f32),
            jax.ShapeDtypeStruct((G, 128), _f32),
        ],
    )(xfull, batch2)


def _tc_apply(xfull, batch2, counts, S1, S2, CNT, w, b, ms, W):
    D = xfull.shape[1]
    Dout = W.shape[1]

    def body(x_ref, b_ref, c_ref, s1_ref, s2_ref, cnt_ref, w_ref, bias_ref,
             ms_ref, W_ref, o_ref):
        cnt = cnt_ref[...][:, :1]
        cntm = jnp.maximum(cnt, 1.0)
        s1 = s1_ref[...]
        mean = s1 / cntm
        t = ms_ref[...] * mean
        sq = s2_ref[...] - 2.0 * t * s1 + cnt * t * t
        std = jnp.sqrt(sq / cntm + EPS)
        A = w_ref[...] / std
        Bb = bias_ref[...] - w_ref[...] * t / std
        C = jnp.dot(Bb, W_ref[...], preferred_element_type=_f32, precision=lax.Precision.HIGHEST)
        oh = _onehot(b_ref[...], R)
        Ar = jnp.dot(oh, A, preferred_element_type=_f32, precision=lax.Precision.HIGHEST)
        Cr = jnp.dot(oh, C, preferred_element_type=_f32, precision=lax.Precision.HIGHEST)
        xw = jnp.dot(Ar * x_ref[...], W_ref[...], preferred_element_type=_f32, precision=lax.Precision.HIGHEST) + Cr
        deg = 1.0 + c_ref[0][:, :1] + c_ref[1][:, :1]
        o_ref[...] = xw * lax.rsqrt(deg)

    return pl.pallas_call(
        body,
        grid=(NB,),
        in_specs=[
            pl.BlockSpec((R, D), lambda i: (i, 0)),
            pl.BlockSpec((R, 1), lambda i: (i, 0)),
            pl.BlockSpec((NC, R, DEGW), lambda i: (0, i, 0)),
            pl.BlockSpec((G, D), lambda i: (0, 0)),
            pl.BlockSpec((G, D), lambda i: (0, 0)),
            pl.BlockSpec((G, 128), lambda i: (0, 0)),
            pl.BlockSpec((1, D), lambda i: (0, 0)),
            pl.BlockSpec((1, D), lambda i: (0, 0)),
            pl.BlockSpec((1, D), lambda i: (0, 0)),
            pl.BlockSpec((D, Dout), lambda i: (0, 0)),
        ],
        out_specs=pl.BlockSpec((R, Dout), lambda i: (i, 0)),
        out_shape=jax.ShapeDtypeStruct((N, Dout), _f32),
    )(xfull, batch2, counts, S1, S2, CNT, w, b, ms, W)


def _tc_epilogue(accp, xws, counts, batch2, bias, want_h):
    def body(a_ref, x_ref, c_ref, b_ref, bias_ref, h_ref, s1_ref, s2_ref):
        i = pl.program_id(0)

        @pl.when(i == 0)
        def _():
            s1_ref[...] = jnp.zeros_like(s1_ref)
            s2_ref[...] = jnp.zeros_like(s2_ref)

        deg = 1.0 + c_ref[0][:, :1] + c_ref[1][:, :1]
        dinv = lax.rsqrt(deg)
        out = dinv * (a_ref[0] + a_ref[1] + x_ref[...]) + bias_ref[...]
        h = _leaky(out)
        h_ref[...] = h
        oh = _onehot(b_ref[...], R)
        dn = (((0,), (0,)), ((), ()))
        s1_ref[...] += lax.dot_general(oh, h, dn, preferred_element_type=_f32, precision=lax.Precision.HIGHEST)
        s2_ref[...] += lax.dot_general(oh, h * h, dn, preferred_element_type=_f32, precision=lax.Precision.HIGHEST)

    h, s1, s2 = pl.pallas_call(
        body,
        grid=(NB,),
        in_specs=[
            pl.BlockSpec((NC, R, DF), lambda i: (0, i, 0)),
            pl.BlockSpec((R, DF), lambda i: (i, 0)),
            pl.BlockSpec((NC, R, DEGW), lambda i: (0, i, 0)),
            pl.BlockSpec((R, 1), lambda i: (i, 0)),
            pl.BlockSpec((1, DF), lambda i: (0, 0)),
        ],
        out_specs=[
            pl.BlockSpec((R, DF), lambda i: (i, 0)),
            pl.BlockSpec((G, DF), lambda i: (0, 0)),
            pl.BlockSpec((G, DF), lambda i: (0, 0)),
        ],
        out_shape=[
            jax.ShapeDtypeStruct((N, DF), _f32),
            jax.ShapeDtypeStruct((G, DF), _f32),
            jax.ShapeDtypeStruct((G, DF), _f32),
        ],
    )(accp, xws, counts, batch2, bias)
    return (h, s1, s2) if want_h else (s1, s2)


def _tc_head(T1, Sh2, CNT, bn1_ga, bn1_ba, bn1_gb, bn1_bb, fc1_Wa, fc1_Wb,
             fc1_b, bn2_g, bn2_b, fc2_W, fc2_b):
    def bn(p, g, b):
        m = jnp.mean(p, axis=0, keepdims=True)
        v = jnp.mean((p - m) * (p - m), axis=0, keepdims=True)
        return (p - m) / jnp.sqrt(v + EPS) * g + b

    def body(t1_ref, sh2_ref, cnt_ref, ga_ref, ba_ref, gb_ref, bb_ref, Wa_ref,
             Wb_ref, fb_ref, g2_ref, b2_ref, W2_ref, f2_ref, y_ref):
        cntm = jnp.maximum(cnt_ref[...][:, :1], 1.0)
        p1 = t1_ref[...] / cntm
        p2 = sh2_ref[...] / cntm
        q = (jnp.dot(bn(p1, ga_ref[...], ba_ref[...]), Wa_ref[...],
                     preferred_element_type=_f32, precision=lax.Precision.HIGHEST)
             + jnp.dot(bn(p2, gb_ref[...], bb_ref[...]), Wb_ref[...],
                       preferred_element_type=_f32, precision=lax.Precision.HIGHEST)
             + fb_ref[...])
        q = _leaky(q)
        q = bn(q, g2_ref[...], b2_ref[...])
        y_ref[...] = jnp.dot(q, W2_ref[...], preferred_element_type=_f32, precision=lax.Precision.HIGHEST) + f2_ref[...]

    return pl.pallas_call(
        body,
        out_shape=jax.ShapeDtypeStruct((G, 1), _f32),
    )(T1, Sh2, CNT, bn1_ga, bn1_ba, bn1_gb, bn1_bb, fc1_Wa, fc1_Wb, fc1_b,
      bn2_g, bn2_b, fc2_W, fc2_b)


def kernel(x, pLDDT, bfactor, edge_index, batch, gn1_w, gn1_b, gn1_ms, W1, b1,
           gn2_w, gn2_b, gn2_ms, W2, b2, bn1_g, bn1_b, fc1_W, fc1_b, bn2_g,
           bn2_b, fc2_W, fc2_b):
    src3 = edge_index[0].reshape(NW, NCHUNK, K)
    dst3 = edge_index[1].reshape(NW, NCHUNK, K)
    batch2 = batch[:, None]

    D1 = 256
    x0p = jnp.concatenate(
        [x, pLDDT[:, None], jnp.zeros((N, D1 - DF - 1), _f32)], axis=1)
    pz = jnp.zeros((D1 - DF - 1,), _f32)
    gn1_wp = jnp.concatenate([gn1_w, pz])[None, :]
    gn1_bp = jnp.concatenate([gn1_b, pz])[None, :]
    gn1_msp = jnp.concatenate([gn1_ms, pz])[None, :]
    W1p = jnp.concatenate([W1, jnp.zeros((D1 - DF - 1, DF), _f32)], axis=0)

    counts = _sc_deg(dst3)

    S1, S2, CNT = _tc_stats(x0p, batch2)
    xws1 = _tc_apply(x0p, batch2, counts, S1, S2, CNT, gn1_wp, gn1_bp, gn1_msp, W1p)

    acc1 = _sc_agg(xws1, src3, dst3)
    h1, T1, T2 = _tc_epilogue(acc1, xws1, counts, batch2, b1[None, :], True)

    xws2 = _tc_apply(h1, batch2, counts, T1, T2, CNT, gn2_w[None, :],
                     gn2_b[None, :], gn2_ms[None, :], W2)

    acc2 = _sc_agg(xws2, src3, dst3)
    Sh2, _ = _tc_epilogue(acc2, xws2, counts, batch2, b2[None, :], False)

    return _tc_head(T1, Sh2, CNT, bn1_g[None, :DF], bn1_b[None, :DF],
                    bn1_g[None, DF:], bn1_b[None, DF:], fc1_W[:DF], fc1_W[DF:],
                    fc1_b[None, :], bn2_g[None, :], bn2_b[None, :], fc2_W,
                    fc2_b[None, :])


# speedup vs baseline: 16.7335x; 1.4396x over previous
import functools

import jax
import jax.numpy as jnp
from jax import lax
from jax.experimental import pallas as pl
from jax.experimental.pallas import tpu as pltpu
from jax.experimental.pallas import tpu_sc as plsc

N = 10000
E = 320000
G = 16
DF = 128
EPS = 1e-5

NC = 2
NS = 16
NW = NC * NS
EPW = E // NW
K = 80
NCHUNK = EPW // K
RPT = 640
WB = 80
DEGW = 8

R = 1000
NB = N // R

_f32 = jnp.float32


def _sc_mesh():
    return plsc.VectorSubcoreMesh(
        core_axis_name="c", subcore_axis_name="s", num_cores=NC, num_subcores=NS)


def _make_sc_deg():
    @functools.partial(
        pl.kernel,
        out_type=jax.ShapeDtypeStruct((NC, N, DEGW), _f32),
        mesh=_sc_mesh(),
        compiler_params=pltpu.CompilerParams(use_tc_tiling_on_sc=False),
        scratch_types=[
            pltpu.VMEM((NCHUNK, K), jnp.int32),
            pltpu.VMEM((K, DEGW), _f32),
            pltpu.VMEM((WB, DEGW), _f32),
            pltpu.VMEM_SHARED((N, DEGW), _f32),
            pltpu.SemaphoreType.DMA,
        ],
    )
    def run(dst_hbm, ones_hbm, zeros_hbm, out_hbm, dstv, ones_v, wbuf, acc, sem):
        c = lax.axis_index("c")
        s = lax.axis_index("s")
        wid = c * NS + s
        nwb = jnp.where(s == NS - 1, 5, 8)
        pltpu.sync_copy(dst_hbm.at[wid], dstv)
        pltpu.sync_copy(ones_hbm, ones_v)
        pltpu.sync_copy(zeros_hbm, wbuf)

        def zacc(k, _):
            pltpu.sync_copy(wbuf, acc.at[pl.ds(s * RPT + k * WB, WB)])
            return 0

        lax.fori_loop(0, nwb, zacc, 0)
        plsc.subcore_barrier()

        def chunk(ci, _):
            pltpu.sync_copy(ones_v, acc.at[dstv.at[ci]], add=True)
            return 0

        lax.fori_loop(0, NCHUNK, chunk, 0)
        plsc.subcore_barrier()

        def wback(k, _):
            rows = pl.ds(s * RPT + k * WB, WB)
            pltpu.sync_copy(acc.at[rows], wbuf)
            pltpu.sync_copy(wbuf, out_hbm.at[c].at[rows])
            return 0

        lax.fori_loop(0, nwb, wback, 0)

    return run


DH = DF // 2


def _make_sc_agg():
    @functools.partial(
        pl.kernel,
        out_type=jax.ShapeDtypeStruct((NC, N, DH), _f32),
        mesh=_sc_mesh(),
        compiler_params=pltpu.CompilerParams(use_tc_tiling_on_sc=False),
        scratch_types=[
            pltpu.VMEM((NCHUNK, K), jnp.int32),
            pltpu.VMEM((NCHUNK, K), jnp.int32),
            pltpu.VMEM((K, DH), _f32),
            pltpu.VMEM((K, DH), _f32),
            pltpu.VMEM((WB, DH), _f32),
            pltpu.VMEM_SHARED((N, DH), _f32),
            pltpu.SemaphoreType.DMA,
            pltpu.SemaphoreType.DMA,
        ],
    )
    def run(xws_hbm, src_hbm, dst_hbm, out_hbm, srcv, dstv, gbuf0, gbuf1, wbuf,
            acc, sem0, sem1):
        c = lax.axis_index("c")
        s = lax.axis_index("s")
        wid = c * NS + s
        nwb = jnp.where(s == NS - 1, 5, 8)
        pltpu.sync_copy(src_hbm.at[wid], srcv)
        pltpu.sync_copy(dst_hbm.at[wid], dstv)

        zero16 = jnp.zeros((16,), _f32)

        def fill_zero(r, _):
            for j in range(DH // 16):
                wbuf[r, pl.ds(j * 16, 16)] = zero16
            return 0

        lax.fori_loop(0, WB, fill_zero, 0)

        def zacc(k, _):
            pltpu.sync_copy(wbuf, acc.at[pl.ds(s * RPT + k * WB, WB)])
            return 0

        lax.fori_loop(0, nwb, zacc, 0)
        plsc.subcore_barrier()

        pltpu.async_copy(xws_hbm.at[srcv.at[0]], gbuf0, sem0)

        def pair(h, _):
            c0 = 2 * h
            pltpu.async_copy(xws_hbm.at[srcv.at[c0 + 1]], gbuf1, sem1)
            pltpu.make_async_copy(xws_hbm.at[srcv.at[c0]], gbuf0, sem0).wait()
            pltpu.sync_copy(gbuf0, acc.at[dstv.at[c0]], add=True)
            pltpu.async_copy(xws_hbm.at[srcv.at[c0 + 2]], gbuf0, sem0)
            pltpu.make_async_copy(xws_hbm.at[srcv.at[c0 + 1]], gbuf1, sem1).wait()
            pltpu.sync_copy(gbuf1, acc.at[dstv.at[c0 + 1]], add=True)
            return 0

        lax.fori_loop(0, (NCHUNK - 1) // 2, pair, 0)
        pltpu.make_async_copy(
            xws_hbm.at[srcv.at[NCHUNK - 1]], gbuf0, sem0).wait()
        pltpu.sync_copy(gbuf0, acc.at[dstv.at[NCHUNK - 1]], add=True)
        plsc.subcore_barrier()

        def wback(k, _):
            rows = pl.ds(s * RPT + k * WB, WB)
            pltpu.sync_copy(acc.at[rows], wbuf)
            pltpu.sync_copy(wbuf, out_hbm.at[c].at[rows])
            return 0

        lax.fori_loop(0, nwb, wback, 0)

    return run


_SC_DEG = _make_sc_deg()
_SC_AGG = _make_sc_agg()


def _sc_deg(dst3):
    return _SC_DEG(dst3, jnp.ones((K, DEGW), _f32), jnp.zeros((WB, DEGW), _f32))


def _sc_agg(xws, src3, dst3):
    lo = _SC_AGG(xws[:, :DH], src3, dst3)
    hi = _SC_AGG(xws[:, DH:], src3, dst3)
    return jnp.concatenate([lo, hi], axis=2)


def _onehot(bb, rows):
    return (bb == lax.broadcasted_iota(jnp.int32, (rows, G), 1)).astype(_f32)


def _leaky(x):
    return jnp.where(x >= 0, x, 0.01 * x)


def _tc_stats(xfull, batch2):
    D = xfull.shape[1]

    def body(x_ref, b_ref, s1_ref, s2_ref, cnt_ref):
        i = pl.program_id(0)

        @pl.when(i == 0)
        def _():
            s1_ref[...] = jnp.zeros_like(s1_ref)
            s2_ref[...] = jnp.zeros_like(s2_ref)
            cnt_ref[...] = jnp.zeros_like(cnt_ref)

        xb = x_ref[...]
        oh = _onehot(b_ref[...], R)
        dn = (((0,), (0,)), ((), ()))
        s1_ref[...] += lax.dot_general(oh, xb, dn, preferred_element_type=_f32, precision=lax.Precision.HIGHEST)
        s2_ref[...] += lax.dot_general(oh, xb * xb, dn, preferred_element_type=_f32, precision=lax.Precision.HIGHEST)
        cnt_ref[...] += jnp.broadcast_to(jnp.sum(oh, axis=0)[:, None], (G, 128))

    return pl.pallas_call(
        body,
        grid=(NB,),
        in_specs=[
            pl.BlockSpec((R, D), lambda i: (i, 0)),
            pl.BlockSpec((R, 1), lambda i: (i, 0)),
        ],
        out_specs=[
            pl.BlockSpec((G, D), lambda i: (0, 0)),
            pl.BlockSpec((G, D), lambda i: (0, 0)),
            pl.BlockSpec((G, 128), lambda i: (0, 0)),
        ],
        out_shape=[
            jax.ShapeDtypeStruct((G, D), _f32),
            jax.ShapeDtypeStruct((G, D), _f32),
            jax.ShapeDtypeStruct((G, 128), _f32),
        ],
    )(xfull, batch2)


def _tc_apply(xfull, batch2, counts, S1, S2, CNT, w, b, ms, W):
    D = xfull.shape[1]
    Dout = W.shape[1]

    def body(x_ref, b_ref, c_ref, s1_ref, s2_ref, cnt_ref, w_ref, bias_ref,
             ms_ref, W_ref, o_ref):
        cnt = cnt_ref[...][:, :1]
        cntm = jnp.maximum(cnt, 1.0)
        s1 = s1_ref[...]
        mean = s1 / cntm
        t = ms_ref[...] * mean
        sq = s2_ref[...] - 2.0 * t * s1 + cnt * t * t
        std = jnp.sqrt(sq / cntm + EPS)
        A = w_ref[...] / std
        Bb = bias_ref[...] - w_ref[...] * t / std
        C = jnp.dot(Bb, W_ref[...], preferred_element_type=_f32, precision=lax.Precision.HIGHEST)
        oh = _onehot(b_ref[...], R)
        Ar = jnp.dot(oh, A, preferred_element_type=_f32, precision=lax.Precision.HIGHEST)
        Cr = jnp.dot(oh, C, preferred_element_type=_f32, precision=lax.Precision.HIGHEST)
        xw = jnp.dot(Ar * x_ref[...], W_ref[...], preferred_element_type=_f32, precision=lax.Precision.HIGHEST) + Cr
        deg = 1.0 + c_ref[0][:, :1] + c_ref[1][:, :1]
        o_ref[...] = xw * lax.rsqrt(deg)

    return pl.pallas_call(
        body,
        grid=(NB,),
        in_specs=[
            pl.BlockSpec((R, D), lambda i: (i, 0)),
            pl.BlockSpec((R, 1), lambda i: (i, 0)),
            pl.BlockSpec((NC, R, DEGW), lambda i: (0, i, 0)),
            pl.BlockSpec((G, D), lambda i: (0, 0)),
            pl.BlockSpec((G, D), lambda i: (0, 0)),
            pl.BlockSpec((G, 128), lambda i: (0, 0)),
            pl.BlockSpec((1, D), lambda i: (0, 0)),
            pl.BlockSpec((1, D), lambda i: (0, 0)),
            pl.BlockSpec((1, D), lambda i: (0, 0)),
            pl.BlockSpec((D, Dout), lambda i: (0, 0)),
        ],
        out_specs=pl.BlockSpec((R, Dout), lambda i: (i, 0)),
        out_shape=jax.ShapeDtypeStruct((N, Dout), _f32),
    )(xfull, batch2, counts, S1, S2, CNT, w, b, ms, W)


def _tc_epilogue(accp, xws, counts, batch2, bias, want_h):
    def body(a_ref, x_ref, c_ref, b_ref, bias_ref, h_ref, s1_ref, s2_ref):
        i = pl.program_id(0)

        @pl.when(i == 0)
        def _():
            s1_ref[...] = jnp.zeros_like(s1_ref)
            s2_ref[...] = jnp.zeros_like(s2_ref)

        deg = 1.0 + c_ref[0][:, :1] + c_ref[1][:, :1]
        dinv = lax.rsqrt(deg)
        out = dinv * (a_ref[0] + a_ref[1] + x_ref[...]) + bias_ref[...]
        h = _leaky(out)
        h_ref[...] = h
        oh = _onehot(b_ref[...], R)
        dn = (((0,), (0,)), ((), ()))
        s1_ref[...] += lax.dot_general(oh, h, dn, preferred_element_type=_f32, precision=lax.Precision.HIGHEST)
        s2_ref[...] += lax.dot_general(oh, h * h, dn, preferred_element_type=_f32, precision=lax.Precision.HIGHEST)

    h, s1, s2 = pl.pallas_call(
        body,
        grid=(NB,),
        in_specs=[
            pl.BlockSpec((NC, R, DF), lambda i: (0, i, 0)),
            pl.BlockSpec((R, DF), lambda i: (i, 0)),
            pl.BlockSpec((NC, R, DEGW), lambda i: (0, i, 0)),
            pl.BlockSpec((R, 1), lambda i: (i, 0)),
            pl.BlockSpec((1, DF), lambda i: (0, 0)),
        ],
        out_specs=[
            pl.BlockSpec((R, DF), lambda i: (i, 0)),
            pl.BlockSpec((G, DF), lambda i: (0, 0)),
            pl.BlockSpec((G, DF), lambda i: (0, 0)),
        ],
        out_shape=[
            jax.ShapeDtypeStruct((N, DF), _f32),
            jax.ShapeDtypeStruct((G, DF), _f32),
            jax.ShapeDtypeStruct((G, DF), _f32),
        ],
    )(accp, xws, counts, batch2, bias)
    return (h, s1, s2) if want_h else (s1, s2)


def _tc_head(T1, Sh2, CNT, bn1_ga, bn1_ba, bn1_gb, bn1_bb, fc1_Wa, fc1_Wb,
             fc1_b, bn2_g, bn2_b, fc2_W, fc2_b):
    def bn(p, g, b):
        m = jnp.mean(p, axis=0, keepdims=True)
        v = jnp.mean((p - m) * (p - m), axis=0, keepdims=True)
        return (p - m) / jnp.sqrt(v + EPS) * g + b

    def body(t1_ref, sh2_ref, cnt_ref, ga_ref, ba_ref, gb_ref, bb_ref, Wa_ref,
             Wb_ref, fb_ref, g2_ref, b2_ref, W2_ref, f2_ref, y_ref):
        cntm = jnp.maximum(cnt_ref[...][:, :1], 1.0)
        p1 = t1_ref[...] / cntm
        p2 = sh2_ref[...] / cntm
        q = (jnp.dot(bn(p1, ga_ref[...], ba_ref[...]), Wa_ref[...],
                     preferred_element_type=_f32, precision=lax.Precision.HIGHEST)
             + jnp.dot(bn(p2, gb_ref[...], bb_ref[...]), Wb_ref[...],
                       preferred_element_type=_f32, precision=lax.Precision.HIGHEST)
             + fb_ref[...])
        q = _leaky(q)
        q = bn(q, g2_ref[...], b2_ref[...])
        y_ref[...] = jnp.dot(q, W2_ref[...], preferred_element_type=_f32, precision=lax.Precision.HIGHEST) + f2_ref[...]

    return pl.pallas_call(
        body,
        out_shape=jax.ShapeDtypeStruct((G, 1), _f32),
    )(T1, Sh2, CNT, bn1_ga, bn1_ba, bn1_gb, bn1_bb, fc1_Wa, fc1_Wb, fc1_b,
      bn2_g, bn2_b, fc2_W, fc2_b)


def kernel(x, pLDDT, bfactor, edge_index, batch, gn1_w, gn1_b, gn1_ms, W1, b1,
           gn2_w, gn2_b, gn2_ms, W2, b2, bn1_g, bn1_b, fc1_W, fc1_b, bn2_g,
           bn2_b, fc2_W, fc2_b):
    src3 = edge_index[0].reshape(NW, NCHUNK, K)
    dst3 = edge_index[1].reshape(NW, NCHUNK, K)
    batch2 = batch[:, None]

    D1 = 256
    x0p = jnp.concatenate(
        [x, pLDDT[:, None], jnp.zeros((N, D1 - DF - 1), _f32)], axis=1)
    pz = jnp.zeros((D1 - DF - 1,), _f32)
    gn1_wp = jnp.concatenate([gn1_w, pz])[None, :]
    gn1_bp = jnp.concatenate([gn1_b, pz])[None, :]
    gn1_msp = jnp.concatenate([gn1_ms, pz])[None, :]
    W1p = jnp.concatenate([W1, jnp.zeros((D1 - DF - 1, DF), _f32)], axis=0)

    counts = _sc_deg(dst3)

    S1, S2, CNT = _tc_stats(x0p, batch2)
    xws1 = _tc_apply(x0p, batch2, counts, S1, S2, CNT, gn1_wp, gn1_bp, gn1_msp, W1p)

    acc1 = _sc_agg(xws1, src3, dst3)
    h1, T1, T2 = _tc_epilogue(acc1, xws1, counts, batch2, b1[None, :], True)

    xws2 = _tc_apply(h1, batch2, counts, T1, T2, CNT, gn2_w[None, :],
                     gn2_b[None, :], gn2_ms[None, :], W2)

    acc2 = _sc_agg(xws2, src3, dst3)
    Sh2, _ = _tc_epilogue(acc2, xws2, counts, batch2, b2[None, :], False)

    return _tc_head(T1, Sh2, CNT, bn1_g[None, :DF], bn1_b[None, :DF],
                    bn1_g[None, DF:], bn1_b[None, DF:], fc1_W[:DF], fc1_W[DF:],
                    fc1_b[None, :], bn2_g[None, :], bn2_b[None, :], fc2_W,
                    fc2_b[None, :])


# speedup vs baseline: 17.9182x; 1.0708x over previous
import functools

import jax
import jax.numpy as jnp
from jax import lax
from jax.experimental import pallas as pl
from jax.experimental.pallas import tpu as pltpu
from jax.experimental.pallas import tpu_sc as plsc

N = 10000
E = 320000
G = 16
DF = 128
EPS = 1e-5

NC = 2
NS = 16
NW = NC * NS
EPW = E // NW
K = 128
NCHUNK = 79
EPAD = NCHUNK * K - EPW
ND = 8
NA = N + ND
RPT = 640
WB = 80
DEGW = 8
DH = DF // 2

R = 1000
NB = N // R

_f32 = jnp.float32
_HI = lax.Precision.HIGHEST


def _sc_mesh():
    return plsc.VectorSubcoreMesh(
        core_axis_name="c", subcore_axis_name="s", num_cores=NC, num_subcores=NS)


def _make_sc_deg():
    @functools.partial(
        pl.kernel,
        out_type=jax.ShapeDtypeStruct((NC, NA, DEGW), _f32),
        mesh=_sc_mesh(),
        compiler_params=pltpu.CompilerParams(use_tc_tiling_on_sc=False),
        scratch_types=[
            pltpu.VMEM((NCHUNK, K), jnp.int32),
            pltpu.VMEM((K, DEGW), _f32),
            pltpu.VMEM((WB, DEGW), _f32),
            pltpu.VMEM_SHARED((NA, DEGW), _f32),
            pltpu.SemaphoreType.DMA,
        ],
    )
    def run(dst_hbm, ones_hbm, zeros_hbm, out_hbm, dstv, ones_v, wbuf, acc, sem):
        c = lax.axis_index("c")
        s = lax.axis_index("s")
        wid = c * NS + s
        nwb = jnp.where(s == NS - 1, 5, 8)
        pltpu.sync_copy(dst_hbm.at[wid], dstv)
        pltpu.sync_copy(ones_hbm, ones_v)
        pltpu.sync_copy(zeros_hbm, wbuf)

        def zacc(k, _):
            pltpu.sync_copy(wbuf, acc.at[pl.ds(s * RPT + k * WB, WB)])
            return 0

        lax.fori_loop(0, nwb, zacc, 0)
        plsc.subcore_barrier()

        def chunk(ci, _):
            pltpu.sync_copy(ones_v, acc.at[dstv.at[ci]], add=True)
            return 0

        lax.fori_loop(0, NCHUNK, chunk, 0)
        plsc.subcore_barrier()

        def wback(k, _):
            rows = pl.ds(s * RPT + k * WB, WB)
            pltpu.sync_copy(acc.at[rows], wbuf)
            pltpu.sync_copy(wbuf, out_hbm.at[c].at[rows])
            return 0

        lax.fori_loop(0, nwb, wback, 0)

    return run


def _make_sc_agg():
    @functools.partial(
        pl.kernel,
        out_type=jax.ShapeDtypeStruct((NC, NA, DH), _f32),
        mesh=_sc_mesh(),
        compiler_params=pltpu.CompilerParams(use_tc_tiling_on_sc=False),
        scratch_types=[
            pltpu.VMEM((NCHUNK, K), jnp.int32),
            pltpu.VMEM((NCHUNK, K), jnp.int32),
            pltpu.VMEM((K, DH), _f32),
            pltpu.VMEM((K, DH), _f32),
            pltpu.VMEM((WB, DH), _f32),
            pltpu.VMEM_SHARED((NA, DH), _f32),
            pltpu.SemaphoreType.DMA,
            pltpu.SemaphoreType.DMA,
        ],
    )
    def run(xws_hbm, src_hbm, dst_hbm, out_hbm, srcv, dstv, gbuf0, gbuf1, wbuf,
            acc, sem0, sem1):
        c = lax.axis_index("c")
        s = lax.axis_index("s")
        wid = c * NS + s
        nwb = jnp.where(s == NS - 1, 5, 8)
        pltpu.sync_copy(src_hbm.at[wid], srcv)
        pltpu.sync_copy(dst_hbm.at[wid], dstv)

        zero16 = jnp.zeros((16,), _f32)

        def fill_zero(r, _):
            for j in range(DH // 16):
                wbuf[r, pl.ds(j * 16, 16)] = zero16
            return 0

        lax.fori_loop(0, WB, fill_zero, 0)

        def zacc(k, _):
            pltpu.sync_copy(wbuf, acc.at[pl.ds(s * RPT + k * WB, WB)])
            return 0

        lax.fori_loop(0, nwb, zacc, 0)
        plsc.subcore_barrier()

        pltpu.async_copy(xws_hbm.at[srcv.at[0]], gbuf0, sem0)

        def pair(h, _):
            c0 = 2 * h
            pltpu.async_copy(xws_hbm.at[srcv.at[c0 + 1]], gbuf1, sem1)
            pltpu.make_async_copy(xws_hbm.at[srcv.at[c0]], gbuf0, sem0).wait()
            pltpu.sync_copy(gbuf0, acc.at[dstv.at[c0]], add=True)
            pltpu.async_copy(xws_hbm.at[srcv.at[c0 + 2]], gbuf0, sem0)
            pltpu.make_async_copy(xws_hbm.at[srcv.at[c0 + 1]], gbuf1, sem1).wait()
            pltpu.sync_copy(gbuf1, acc.at[dstv.at[c0 + 1]], add=True)
            return 0

        lax.fori_loop(0, (NCHUNK - 1) // 2, pair, 0)
        pltpu.make_async_copy(
            xws_hbm.at[srcv.at[NCHUNK - 1]], gbuf0, sem0).wait()
        pltpu.sync_copy(gbuf0, acc.at[dstv.at[NCHUNK - 1]], add=True)
        plsc.subcore_barrier()

        def wback(k, _):
            rows = pl.ds(s * RPT + k * WB, WB)
            pltpu.sync_copy(acc.at[rows], wbuf)
            pltpu.sync_copy(wbuf, out_hbm.at[c].at[rows])
            return 0

        lax.fori_loop(0, nwb, wback, 0)

    return run


_SC_CACHE = {}


def _sc_deg(dst3):
    if "deg" not in _SC_CACHE:
        _SC_CACHE["deg"] = _make_sc_deg()
    out = _SC_CACHE["deg"](
        dst3, jnp.ones((K, DEGW), _f32), jnp.zeros((WB, DEGW), _f32))
    return out[:, :N]


def _sc_agg(xws, src3, dst3):
    if "agg" not in _SC_CACHE:
        _SC_CACHE["agg"] = _make_sc_agg()
    lo = _SC_CACHE["agg"](xws[:, :DH], src3, dst3)
    hi = _SC_CACHE["agg"](xws[:, DH:], src3, dst3)
    return jnp.concatenate([lo[:, :N], hi[:, :N]], axis=2)


def _onehot(bb, rows):
    return (bb == lax.broadcasted_iota(jnp.int32, (rows, G), 1)).astype(_f32)


def _leaky(x):
    return jnp.where(x >= 0, x, 0.01 * x)


def _dotg(a, b):
    return lax.dot_general(a, b, (((0,), (0,)), ((), ())),
                           preferred_element_type=_f32, precision=_HI)


def _gn_coeffs(s1_ref, s2_ref, cnt_ref, w_ref, bias_ref, ms_ref):
    cnt = cnt_ref[...][:, :1]
    cntm = jnp.maximum(cnt, 1.0)
    s1 = s1_ref[...]
    mean = s1 / cntm
    t = ms_ref[...] * mean
    sq = s2_ref[...] - 2.0 * t * s1 + cnt * t * t
    std = jnp.sqrt(sq / cntm + EPS)
    A = w_ref[...] / std
    B = bias_ref[...] - w_ref[...] * t / std
    return A, B


def _dinv_of(c_ref):
    return lax.rsqrt(1.0 + c_ref[0][:, :1] + c_ref[1][:, :1])


def _tc_layer1(x0p, batch2, counts, w, b, ms, W):
    D = x0p.shape[1]
    Dout = W.shape[1]

    def body(x_ref, b_ref, c_ref, w_ref, bias_ref, ms_ref, W_ref, o_ref,
             s1, s2, cnt):
        p = pl.program_id(0)
        i = pl.program_id(1)

        @pl.when((p == 0) & (i == 0))
        def _():
            s1[...] = jnp.zeros_like(s1)
            s2[...] = jnp.zeros_like(s2)
            cnt[...] = jnp.zeros_like(cnt)

        oh = _onehot(b_ref[...], R)

        @pl.when(p == 0)
        def _():
            xb = x_ref[...]
            s1[...] += _dotg(oh, xb)
            s2[...] += _dotg(oh, xb * xb)
            cnt[...] += jnp.broadcast_to(jnp.sum(oh, axis=0)[:, None], (G, 128))

        @pl.when(p == 1)
        def _():
            A, B = _gn_coeffs(s1, s2, cnt, w_ref, bias_ref, ms_ref)
            C = jnp.dot(B, W_ref[...], preferred_element_type=_f32, precision=_HI)
            Ar = jnp.dot(oh, A, preferred_element_type=_f32, precision=_HI)
            Cr = jnp.dot(oh, C, preferred_element_type=_f32, precision=_HI)
            xw = jnp.dot(Ar * x_ref[...], W_ref[...],
                         preferred_element_type=_f32, precision=_HI) + Cr
            o_ref[...] = xw * _dinv_of(c_ref)

    return pl.pallas_call(
        body,
        grid=(2, NB),
        in_specs=[
            pl.BlockSpec((R, D), lambda p, i: (i, 0)),
            pl.BlockSpec((R, 1), lambda p, i: (i, 0)),
            pl.BlockSpec((NC, R, DEGW), lambda p, i: (0, i, 0)),
            pl.BlockSpec((1, D), lambda p, i: (0, 0)),
            pl.BlockSpec((1, D), lambda p, i: (0, 0)),
            pl.BlockSpec((1, D), lambda p, i: (0, 0)),
            pl.BlockSpec((D, Dout), lambda p, i: (0, 0)),
        ],
        out_specs=pl.BlockSpec((R, Dout), lambda p, i: (p * i, 0)),
        out_shape=jax.ShapeDtypeStruct((N, Dout), _f32),
        scratch_shapes=[
            pltpu.VMEM((G, D), _f32),
            pltpu.VMEM((G, D), _f32),
            pltpu.VMEM((G, 128), _f32),
        ],
    )(x0p, batch2, counts, w, b, ms, W)


def _tc_layer2(accp, xws1, counts, batch2, b1, w, b, ms, W):
    def body(a_ref, x_ref, c_ref, b_ref, b1_ref, w_ref, bias_ref, ms_ref, W_ref,
             o_ref, t1_ref, h1s, s1, s2, cnt):
        p = pl.program_id(0)
        i = pl.program_id(1)

        @pl.when((p == 0) & (i == 0))
        def _():
            s1[...] = jnp.zeros_like(s1)
            s2[...] = jnp.zeros_like(s2)
            cnt[...] = jnp.zeros_like(cnt)

        oh = _onehot(b_ref[...], R)

        @pl.when(p == 0)
        def _():
            out1 = _dinv_of(c_ref) * (a_ref[0] + a_ref[1] + x_ref[...]) + b1_ref[...]
            h = _leaky(out1)
            h1s[pl.ds(i * R, R), :] = h
            s1[...] += _dotg(oh, h)
            s2[...] += _dotg(oh, h * h)
            cnt[...] += jnp.broadcast_to(jnp.sum(oh, axis=0)[:, None], (G, 128))

        @pl.when(p == 1)
        def _():
            A, B = _gn_coeffs(s1, s2, cnt, w_ref, bias_ref, ms_ref)
            C = jnp.dot(B, W_ref[...], preferred_element_type=_f32, precision=_HI)
            Ar = jnp.dot(oh, A, preferred_element_type=_f32, precision=_HI)
            Cr = jnp.dot(oh, C, preferred_element_type=_f32, precision=_HI)
            hb = h1s[pl.ds(i * R, R), :]
            xw = jnp.dot(Ar * hb, W_ref[...],
                         preferred_element_type=_f32, precision=_HI) + Cr
            o_ref[...] = xw * _dinv_of(c_ref)
            t1_ref[...] = s1[...]

    return pl.pallas_call(
        body,
        grid=(2, NB),
        in_specs=[
            pl.BlockSpec((NC, R, DF), lambda p, i: (0, i, 0)),
            pl.BlockSpec((R, DF), lambda p, i: (i, 0)),
            pl.BlockSpec((NC, R, DEGW), lambda p, i: (0, i, 0)),
            pl.BlockSpec((R, 1), lambda p, i: (i, 0)),
            pl.BlockSpec((1, DF), lambda p, i: (0, 0)),
            pl.BlockSpec((1, DF), lambda p, i: (0, 0)),
            pl.BlockSpec((1, DF), lambda p, i: (0, 0)),
            pl.BlockSpec((1, DF), lambda p, i: (0, 0)),
            pl.BlockSpec((DF, DF), lambda p, i: (0, 0)),
        ],
        out_specs=[
            pl.BlockSpec((R, DF), lambda p, i: (p * i, 0)),
            pl.BlockSpec((G, DF), lambda p, i: (0, 0)),
        ],
        out_shape=[
            jax.ShapeDtypeStruct((N, DF), _f32),
            jax.ShapeDtypeStruct((G, DF), _f32),
        ],
        scratch_shapes=[
            pltpu.VMEM((N, DF), _f32),
            pltpu.VMEM((G, DF), _f32),
            pltpu.VMEM((G, DF), _f32),
            pltpu.VMEM((G, 128), _f32),
        ],
    )(accp, xws1, counts, batch2, b1, w, b, ms, W)


def _tc_final(accp, xws2, counts, batch2, b2, T1, bn1_ga, bn1_ba, bn1_gb, bn1_bb,
              fc1_Wa, fc1_Wb, fc1_b, bn2_g, bn2_b, fc2_W, fc2_b):
    def bn(pv, g, b):
        m = jnp.mean(pv, axis=0, keepdims=True)
        v = jnp.mean((pv - m) * (pv - m), axis=0, keepdims=True)
        return (pv - m) / jnp.sqrt(v + EPS) * g + b

    def body(a_ref, x_ref, c_ref, b_ref, b2_ref, t1_ref, ga_ref, ba_ref, gb_ref,
             bb_ref, Wa_ref, Wb_ref, fb_ref, g2_ref, be2_ref, W2_ref, f2_ref,
             y_ref, sh2, cnt):
        i = pl.program_id(0)

        @pl.when(i == 0)
        def _():
            sh2[...] = jnp.zeros_like(sh2)
            cnt[...] = jnp.zeros_like(cnt)

        out2 = _dinv_of(c_ref) * (a_ref[0] + a_ref[1] + x_ref[...]) + b2_ref[...]
        h2 = _leaky(out2)
        oh = _onehot(b_ref[...], R)
        sh2[...] += _dotg(oh, h2)
        cnt[...] += jnp.broadcast_to(jnp.sum(oh, axis=0)[:, None], (G, 128))

        @pl.when(i == NB - 1)
        def _():
            cntm = jnp.maximum(cnt[...][:, :1], 1.0)
            p1 = t1_ref[...] / cntm
            p2 = sh2[...] / cntm
            q = (jnp.dot(bn(p1, ga_ref[...], ba_ref[...]), Wa_ref[...],
                         preferred_element_type=_f32, precision=_HI)
                 + jnp.dot(bn(p2, gb_ref[...], bb_ref[...]), Wb_ref[...],
                           preferred_element_type=_f32, precision=_HI)
                 + fb_ref[...])
            q = _leaky(q)
            q = bn(q, g2_ref[...], be2_ref[...])
            y_ref[...] = jnp.dot(q, W2_ref[...],
                                 preferred_element_type=_f32, precision=_HI) + f2_ref[...]

    return pl.pallas_call(
        body,
        grid=(NB,),
        in_specs=[
            pl.BlockSpec((NC, R, DF), lambda i: (0, i, 0)),
            pl.BlockSpec((R, DF), lambda i: (i, 0)),
            pl.BlockSpec((NC, R, DEGW), lambda i: (0, i, 0)),
            pl.BlockSpec((R, 1), lambda i: (i, 0)),
            pl.BlockSpec((1, DF), lambda i: (0, 0)),
            pl.BlockSpec((G, DF), lambda i: (0, 0)),
            pl.BlockSpec((1, DF), lambda i: (0, 0)),
            pl.BlockSpec((1, DF), lambda i: (0, 0)),
            pl.BlockSpec((1, DF), lambda i: (0, 0)),
            pl.BlockSpec((1, DF), lambda i: (0, 0)),
            pl.BlockSpec((DF, DF), lambda i: (0, 0)),
            pl.BlockSpec((DF, DF), lambda i: (0, 0)),
            pl.BlockSpec((1, DF), lambda i: (0, 0)),
            pl.BlockSpec((1, DF), lambda i: (0, 0)),
            pl.BlockSpec((1, DF), lambda i: (0, 0)),
            pl.BlockSpec((DF, 1), lambda i: (0, 0)),
            pl.BlockSpec((1, 1), lambda i: (0, 0)),
        ],
        out_specs=pl.BlockSpec((G, 1), lambda i: (0, 0)),
        out_shape=jax.ShapeDtypeStruct((G, 1), _f32),
        scratch_shapes=[
            pltpu.VMEM((G, DF), _f32),
            pltpu.VMEM((G, 128), _f32),
        ],
    )(accp, xws2, counts, batch2, b2, T1, bn1_ga, bn1_ba, bn1_gb, bn1_bb,
      fc1_Wa, fc1_Wb, fc1_b, bn2_g, bn2_b, fc2_W, fc2_b)


def kernel(x, pLDDT, bfactor, edge_index, batch, gn1_w, gn1_b, gn1_ms, W1, b1,
           gn2_w, gn2_b, gn2_ms, W2, b2, bn1_g, bn1_b, fc1_W, fc1_b, bn2_g,
           bn2_b, fc2_W, fc2_b):
    src = edge_index[0].reshape(NW, EPW)
    dst = edge_index[1].reshape(NW, EPW)
    pad_s = (jnp.arange(NW * EPAD, dtype=jnp.int32) * 131) % N
    pad_d = N + (jnp.arange(NW * EPAD, dtype=jnp.int32) % ND)
    src3 = jnp.concatenate([src, pad_s.reshape(NW, EPAD)], 1).reshape(NW, NCHUNK, K)
    dst3 = jnp.concatenate([dst, pad_d.reshape(NW, EPAD)], 1).reshape(NW, NCHUNK, K)
    batch2 = batch[:, None]

    D1 = 256
    x0p = jnp.concatenate(
        [x, pLDDT[:, None], jnp.zeros((N, D1 - DF - 1), _f32)], axis=1)
    pz = jnp.zeros((D1 - DF - 1,), _f32)
    gn1_wp = jnp.concatenate([gn1_w, pz])[None, :]
    gn1_bp = jnp.concatenate([gn1_b, pz])[None, :]
    gn1_msp = jnp.concatenate([gn1_ms, pz])[None, :]
    W1p = jnp.concatenate([W1, jnp.zeros((D1 - DF - 1, DF), _f32)], axis=0)

    counts = _sc_deg(dst3)

    xws1 = _tc_layer1(x0p, batch2, counts, gn1_wp, gn1_bp, gn1_msp, W1p)
    acc1 = _sc_agg(xws1, src3, dst3)
    xws2, T1 = _tc_layer2(acc1, xws1, counts, batch2, b1[None, :],
                          gn2_w[None, :], gn2_b[None, :], gn2_ms[None, :], W2)
    acc2 = _sc_agg(xws2, src3, dst3)
    return _tc_final(acc2, xws2, counts, batch2, b2[None, :], T1,
                     bn1_g[None, :DF], bn1_b[None, :DF], bn1_g[None, DF:],
                     bn1_b[None, DF:], fc1_W[:DF], fc1_W[DF:], fc1_b[None, :],
                     bn2_g[None, :], bn2_b[None, :], fc2_W, fc2_b[None, :])
